# baseline (speedup 1.0000x reference)
import functools

import jax
import jax.numpy as jnp
import numpy as np
from jax.experimental import pallas as pl
from jax.experimental.pallas import tpu as pltpu

_N = 8192
_N1 = 64
_N2 = 128


def _make_tables():
    a64 = np.arange(_N1, dtype=np.float64)
    a128 = np.arange(_N2, dtype=np.float64)
    ph64 = -2.0 * np.pi * np.outer(a64, a64) / _N1
    pht = -2.0 * np.pi * np.outer(a128, a64) / _N
    ph128 = -2.0 * np.pi * np.outer(a128, a128) / _N2
    pht2 = -2.0 * np.pi * np.outer(a64, a128) / _N
    w64r, w64i = np.cos(ph64), np.sin(ph64)
    tr_, ti_ = np.cos(pht), np.sin(pht)
    w128r, w128i = np.cos(ph128), np.sin(ph128)
    w64cat = np.concatenate([w64r, w64i], axis=1)
    t1cat = np.concatenate([tr_, tr_], axis=1)
    t2cat = np.concatenate([-ti_, ti_], axis=1)
    w128cat = np.concatenate([w128r[:, :64], w128i[:, :64]], axis=1)
    f32 = np.float32
    return (
        w64r.astype(f32), w64i.astype(f32),
        w128r.astype(f32), w128i.astype(f32),
        np.cos(pht2).astype(f32), np.sin(pht2).astype(f32),
        w64cat.astype(f32), t1cat.astype(f32), t2cat.astype(f32),
        w128cat.astype(f32),
    )


_TABLES = _make_tables()

_HP = jax.lax.Precision.HIGHEST


def _mm(a, b, precision=_HP):
    return jax.lax.dot_general(
        a, b, (((1,), (0,)), ((), ())),
        precision=precision, preferred_element_type=jnp.float32)


def _fused_kernel(x_ref, te_ref, w_ref, b_ref,
                  w128r_ref, w128i_ref, t2r_ref, t2i_ref,
                  w64catr_ref, t1cat_ref, t2cat_ref, w128cat_ref,
                  w64r_ref, w64i_ref, o_ref):
    R = x_ref.shape[1]
    s = jax.lax.dot_general(
        te_ref[0], w_ref[...], (((1,), (1,)), ((), ())),
        precision=_HP, preferred_element_type=jnp.float32) + b_ref[0]
    mod = 1.0 + jnp.tanh(s)
    xm = x_ref[0] * mod.reshape(R, 1)

    w64r = w64r_ref[...]
    w64i = w64i_ref[...]
    w128r = w128r_ref[...]
    w128i = w128i_ref[...]

    at = jnp.swapaxes(xm.reshape(R, _N1, _N2), 1, 2).reshape(R * _N2, _N1)
    b3 = _mm(at, w64catr_ref[...]).reshape(R, _N2, _N2)
    broll = jnp.roll(b3, _N1, axis=2)
    c3 = b3 * t1cat_ref[...] + broll * t2cat_ref[...]
    ct = jnp.swapaxes(c3, 1, 2)
    crt = ct[:, :_N1, :].reshape(R * _N1, _N2)
    cit = ct[:, _N1:, :].reshape(R * _N1, _N2)
    p = _mm(crt, w128cat_ref[...]).reshape(R, _N1, _N2)
    q = _mm(cit, w128cat_ref[...]).reshape(R, _N1, _N2)
    qroll = jnp.roll(q, _N1, axis=2)
    xr3 = p - qroll
    xi3m = p + qroll
    xi3 = jnp.roll(xi3m, _N1, axis=2)
    sgn = 1.0 - 2.0 * (jax.lax.broadcasted_iota(
        jnp.int32, (1, _N2, 1), 1) % 2).astype(jnp.float32)
    x4 = jnp.sum(c3 * sgn, axis=1)
    lane128 = jax.lax.broadcasted_iota(jnp.int32, (1, _N2), 1)
    xr4 = jnp.sum(jnp.where(lane128 == 0, x4, 0.0), axis=1, keepdims=True)
    xi4 = jnp.sum(jnp.where(lane128 == _N1, x4, 0.0), axis=1, keepdims=True)
    amp4 = (xr4 * xr4 + xi4 * xi4).reshape(R, 1, 1)

    sub = jax.lax.broadcasted_iota(jnp.int32, (1, _N1, _N2), 1)
    lane = jax.lax.broadcasted_iota(jnp.int32, (1, _N1, _N2), 2)
    kb3 = sub + _N1 * lane
    amp = jnp.where(lane < _N1, xr3 * xr3 + xi3 * xi3, -1.0)

    ampw = amp
    a4w = amp4
    mv = jnp.maximum(jnp.max(ampw, axis=(1, 2), keepdims=True), a4w)
    for _ in range(7):
        ampw = jnp.where(ampw == mv, -2.0, ampw)
        a4w = jnp.where(a4w == mv, -2.0, a4w)
        mv = jnp.maximum(jnp.max(ampw, axis=(1, 2), keepdims=True), a4w)
    v8 = mv
    ampw = jnp.where(ampw == mv, -2.0, ampw)
    a4w = jnp.where(a4w == mv, -2.0, a4w)
    v9 = jnp.maximum(jnp.max(ampw, axis=(1, 2), keepdims=True), a4w)
    thr = 0.5 * v8 + 0.5 * v9
    keep = amp > thr
    keep4 = amp4 > thr

    wmask = jnp.where(kb3 == 0, 1.0, 2.0)
    nyq = kb3 == _N // 2
    a2r = jnp.where(keep, xr3, 0.0) * wmask + jnp.where(
        nyq & keep4, xr4.reshape(R, 1, 1), 0.0)
    a2i = jnp.where(keep, xi3, 0.0) * wmask + jnp.where(
        nyq & keep4, xi4.reshape(R, 1, 1), 0.0)

    t2r = t2r_ref[...]
    t2i = t2i_ref[...]
    hi = jax.lax.Precision.DEFAULT
    a2r = a2r.reshape(R * _N1, _N2)
    a2i = a2i.reshape(R * _N1, _N2)
    q1 = _mm(a2r, w128r, hi)
    q2 = -_mm(a2i, w128i, hi)
    q3 = _mm(a2r + a2i, w128r - w128i, hi)
    b2r = (q1 - q2).reshape(R, _N1, _N2)
    b2i = (q3 - q1 - q2).reshape(R, _N1, _N2)
    c2r = b2r * t2r + b2i * t2i
    c2i = b2i * t2r - b2r * t2i
    c2rt = jnp.swapaxes(c2r, 1, 2).reshape(R * _N2, _N1)
    c2it = jnp.swapaxes(c2i, 1, 2).reshape(R * _N2, _N1)
    y = (_mm(c2rt, w64r, hi) + _mm(c2it, w64i, hi)).reshape(R, _N2, _N1)
    y = jnp.swapaxes(y, 1, 2).reshape(R, _N) * (1.0 / _N)
    o_ref[...] = y.reshape(1, R, _N).astype(o_ref.dtype)


@functools.partial(jax.jit, static_argnames=())
def kernel(x, time_emb, W, b):
    B, C, N = x.shape
    assert N == _N, "kernel specialized to N=8192"
    R = min(64, C)
    grid = (B, C // R)
    b2 = b.reshape(C // R, 1, R).astype(jnp.float32)
    te3 = time_emb.reshape(B, 1, 256).astype(jnp.float32)
    (w64r, w64i, w128r, w128i, t2r, t2i,
     w64cat, t1cat, t2cat, w128cat) = [jnp.asarray(t) for t in _TABLES]

    def _full(a):
        shape = a.shape
        return pl.BlockSpec(shape, lambda i, j: (0,) * len(shape))

    out = pl.pallas_call(
        _fused_kernel,
        grid=grid,
        in_specs=[
            pl.BlockSpec((1, R, _N), lambda i, j: (i, j, 0)),
            pl.BlockSpec((1, 1, 256), lambda i, j: (i, 0, 0)),
            pl.BlockSpec((R, 256), lambda i, j: (j, 0)),
            pl.BlockSpec((1, 1, R), lambda i, j: (j, 0, 0)),
            _full(w128r), _full(w128i), _full(t2r), _full(t2i),
            _full(w64cat), _full(t1cat), _full(t2cat), _full(w128cat),
            _full(w64r), _full(w64i),
        ],
        out_specs=pl.BlockSpec((1, R, _N), lambda i, j: (i, j, 0)),
        out_shape=jax.ShapeDtypeStruct((B, C, N), x.dtype),
        compiler_params=pltpu.CompilerParams(
            dimension_semantics=("parallel", "parallel")),
    )(x.astype(jnp.float32), te3, W.astype(jnp.float32), b2,
      w128r, w128i, t2r, t2i, w64cat, t1cat, t2cat, w128cat, w64r, w64i)
    return out


# speedup vs baseline: 1.0209x; 1.0209x over previous
import functools

import jax
import jax.numpy as jnp
import numpy as np
from jax.experimental import pallas as pl
from jax.experimental.pallas import tpu as pltpu

_N = 8192
_N1 = 64
_N2 = 128


def _make_tables():
    a64 = np.arange(_N1, dtype=np.float64)
    a128 = np.arange(_N2, dtype=np.float64)
    ph64 = -2.0 * np.pi * np.outer(a64, a64) / _N1
    pht = -2.0 * np.pi * np.outer(a128, a64) / _N
    ph128 = -2.0 * np.pi * np.outer(a128, a128) / _N2
    pht2 = -2.0 * np.pi * np.outer(a64, a128) / _N
    return (
        np.cos(ph64).astype(np.float32), np.sin(ph64).astype(np.float32),
        np.cos(pht).astype(np.float32), np.sin(pht).astype(np.float32),
        np.cos(ph128).astype(np.float32), np.sin(ph128).astype(np.float32),
        np.cos(pht2).astype(np.float32), np.sin(pht2).astype(np.float32),
    )


_TABLES = _make_tables()

_HP = jax.lax.Precision.HIGHEST


def _mm(a, b, precision=_HP):
    return jax.lax.dot_general(
        a, b, (((1,), (0,)), ((), ())),
        precision=precision, preferred_element_type=jnp.float32)


def _fused_kernel(x_ref, te_ref, w_ref, b_ref,
                  w64r_ref, w64i_ref, tr_ref, ti_ref, w128r_ref, w128i_ref,
                  t2r_ref, t2i_ref, o_ref):
    R = x_ref.shape[1]
    s = jax.lax.dot_general(
        te_ref[0], w_ref[...], (((1,), (1,)), ((), ())),
        precision=_HP, preferred_element_type=jnp.float32) + b_ref[0]
    mod = 1.0 + jnp.tanh(s)
    xm = x_ref[0] * mod.reshape(R, 1)

    w64r = w64r_ref[...]
    w64i = w64i_ref[...]
    tr = tr_ref[...]
    ti = ti_ref[...]
    w128r = w128r_ref[...]
    w128i = w128i_ref[...]

    at = jnp.swapaxes(xm.reshape(R, _N1, _N2), 1, 2).reshape(R * _N2, _N1)
    br = _mm(at, w64r).reshape(R, _N2, _N1)
    bi = _mm(at, w64i).reshape(R, _N2, _N1)
    cr = br * tr - bi * ti
    ci = br * ti + bi * tr
    crt = jnp.swapaxes(cr, 1, 2).reshape(R * _N1, _N2)
    cit = jnp.swapaxes(ci, 1, 2).reshape(R * _N1, _N2)
    p1 = _mm(crt, w128r)
    p2 = _mm(cit, w128i)
    p3 = _mm(crt + cit, w128r + w128i)
    xr = (p1 - p2).reshape(R, _N)
    xi = (p3 - p1 - p2).reshape(R, _N)

    pf = jax.lax.broadcasted_iota(jnp.int32, (1, _N), 1)
    kb = (pf // _N2) + _N1 * (pf % _N2)
    amp = xr * xr + xi * xi
    amp = jnp.where(kb <= _N // 2, amp, -1.0)

    ampw = amp
    mv = jnp.max(ampw, axis=1, keepdims=True)
    for _ in range(7):
        ampw = jnp.where(ampw == mv, -2.0, ampw)
        mv = jnp.max(ampw, axis=1, keepdims=True)
    v8 = mv
    ampw = jnp.where(ampw == mv, -2.0, ampw)
    v9 = jnp.max(ampw, axis=1, keepdims=True)
    keep = amp > 0.5 * v8 + 0.5 * v9

    wmask = jnp.where((kb == 0) | (kb == _N // 2), 1.0, 2.0)
    a2r = (jnp.where(keep, xr, 0.0) * wmask).reshape(R, _N1, _N2)
    a2i = (jnp.where(keep, xi, 0.0) * wmask).reshape(R, _N1, _N2)

    t2r = t2r_ref[...]
    t2i = t2i_ref[...]
    hi = jax.lax.Precision.DEFAULT
    a2r = a2r.reshape(R * _N1, _N2)
    a2i = a2i.reshape(R * _N1, _N2)
    q1 = _mm(a2r, w128r, hi)
    q2 = -_mm(a2i, w128i, hi)
    q3 = _mm(a2r + a2i, w128r - w128i, hi)
    b2r = (q1 - q2).reshape(R, _N1, _N2)
    b2i = (q3 - q1 - q2).reshape(R, _N1, _N2)
    c2r = b2r * t2r + b2i * t2i
    c2i = b2i * t2r - b2r * t2i
    c2rt = jnp.swapaxes(c2r, 1, 2).reshape(R * _N2, _N1)
    c2it = jnp.swapaxes(c2i, 1, 2).reshape(R * _N2, _N1)
    y = (_mm(c2rt, w64r, hi) + _mm(c2it, w64i, hi)).reshape(R, _N2, _N1)
    y = jnp.swapaxes(y, 1, 2).reshape(R, _N) * (1.0 / _N)
    o_ref[...] = y.reshape(1, R, _N).astype(o_ref.dtype)


@functools.partial(jax.jit, static_argnames=())
def kernel(x, time_emb, W, b):
    B, C, N = x.shape
    assert N == _N, "kernel specialized to N=8192"
    R = min(64, C)
    grid = (B, C // R)
    b2 = b.reshape(C // R, 1, R).astype(jnp.float32)
    te3 = time_emb.reshape(B, 1, 256).astype(jnp.float32)
    tabs = [jnp.asarray(t) for t in _TABLES]
    out = pl.pallas_call(
        _fused_kernel,
        grid=grid,
        in_specs=[
            pl.BlockSpec((1, R, _N), lambda i, j: (i, j, 0)),
            pl.BlockSpec((1, 1, 256), lambda i, j: (i, 0, 0)),
            pl.BlockSpec((R, 256), lambda i, j: (j, 0)),
            pl.BlockSpec((1, 1, R), lambda i, j: (j, 0, 0)),
            pl.BlockSpec((_N1, _N1), lambda i, j: (0, 0)),
            pl.BlockSpec((_N1, _N1), lambda i, j: (0, 0)),
            pl.BlockSpec((_N2, _N1), lambda i, j: (0, 0)),
            pl.BlockSpec((_N2, _N1), lambda i, j: (0, 0)),
            pl.BlockSpec((_N2, _N2), lambda i, j: (0, 0)),
            pl.BlockSpec((_N2, _N2), lambda i, j: (0, 0)),
            pl.BlockSpec((_N1, _N2), lambda i, j: (0, 0)),
            pl.BlockSpec((_N1, _N2), lambda i, j: (0, 0)),
        ],
        out_specs=pl.BlockSpec((1, R, _N), lambda i, j: (i, j, 0)),
        out_shape=jax.ShapeDtypeStruct((B, C, N), x.dtype),
        compiler_params=pltpu.CompilerParams(
            dimension_semantics=("parallel", "parallel")),
    )(x.astype(jnp.float32), te3, W.astype(jnp.float32), b2, *tabs)
    return out
